# baseline (speedup 1.0000x reference)
import functools

import jax
import jax.numpy as jnp
from jax import lax
from jax.experimental import pallas as pl
from jax.experimental.pallas import tpu as pltpu, tpu_sc as plsc

_RING = 9
_ICHUNK = 128


def _user_lookup_sorted(sidx, dlist, starts, ends, cntb, tab_t, B, D, NW,
                        num_cores):
    b_per_w = B // NW
    nvec = b_per_w // 16
    sidx3 = jnp.reshape(sidx, (NW, nvec, 16))
    dlist3 = jnp.reshape(dlist, (NW, nvec, 16))
    starts3 = jnp.reshape(starts, (NW, nvec, 16))
    ends3 = jnp.reshape(ends, (NW, nvec, 16))
    ntile_max = (tab_t.shape[1] + 127) // 128 - 1
    mesh = plsc.VectorSubcoreMesh(core_axis_name="c", subcore_axis_name="s")

    @functools.partial(
        pl.kernel,
        mesh=mesh,
        out_type=jax.ShapeDtypeStruct((B * D,), jnp.float32),
        scratch_types=[
            pltpu.VMEM((nvec, 16), jnp.int32),
            pltpu.VMEM((nvec, 16), jnp.int32),
            pltpu.VMEM((nvec, 16), jnp.int32),
            pltpu.VMEM((nvec, 16), jnp.int32),
            pltpu.VMEM((16,), jnp.int32),
            pltpu.VMEM((_RING, D, 128), jnp.float32),
            pltpu.VMEM((b_per_w * D,), jnp.float32),
        ] + [pltpu.SemaphoreType.DMA] * _RING,
        compiler_params=pltpu.CompilerParams(use_tc_tiling_on_sc=True,
                                             needs_layout_passes=False),
    )
    def k(sidx_hbm, dl_hbm, st_hbm, en_hbm, cnt_hbm, tab_hbm, out_hbm,
          sidx_v, dl_v, st_v, en_v, cnt_v, ring_v, out_v, *sems):
        wid = lax.axis_index("s") * num_cores + lax.axis_index("c")
        pltpu.sync_copy(sidx_hbm.at[wid], sidx_v)
        pltpu.sync_copy(dl_hbm.at[wid], dl_v)
        pltpu.sync_copy(st_hbm.at[wid], st_v)
        pltpu.sync_copy(en_hbm.at[wid], en_v)
        pltpu.sync_copy(cnt_hbm.at[wid], cnt_v)
        lanes = lax.iota(jnp.int32, 16)

        def lane_of(vec, lane):
            return jnp.max(jnp.where(lanes == lane, vec, 0))

        def dynload(ref, j):
            vec = ref[j >> 4, pl.ds(0, 16)]
            return lane_of(vec, j & 15)

        cnt = lane_of(cnt_v[pl.ds(0, 16)], 0)

        def fetch(b, d):
            dc = jnp.minimum(d, cnt - 1)
            t = jnp.minimum(dynload(dl_v, dc), ntile_max)
            off = pl.multiple_of(t * 128, 128)
            pltpu.async_copy(tab_hbm.at[:, pl.ds(off, 128)], ring_v.at[b],
                             sems[b])

        for b in range(_RING):
            fetch(b, jnp.int32(b))

        def gbody(g, carry):
            for b in range(_RING):
                d = g * _RING + b
                pltpu.make_async_copy(tab_hbm.at[:, pl.ds(0, 128)],
                                      ring_v.at[b], sems[b]).wait()
                dc = jnp.minimum(d, cnt - 1)
                s = dynload(st_v, dc)
                e = dynload(en_v, dc)

                def ebody(j, car):
                    iv = dynload(sidx_v, j)
                    col = iv & 127
                    for kq in range(D // 16):
                        dvec = lanes + kq * 16
                        vals = plsc.load_gather(
                            ring_v.at[b],
                            [dvec, jnp.full((16,), 1, jnp.int32) * col])
                        plsc.store_scatter(
                            out_v, [lanes + (j * D + kq * 16)], vals)
                    return car

                lax.fori_loop(s, e, ebody, 0)
                fetch(b, (g + 1) * _RING + b)
            return carry

        ngroups = (cnt + _RING - 1) // _RING
        lax.fori_loop(0, ngroups, gbody, 0)
        for b in range(_RING):
            pltpu.make_async_copy(tab_hbm.at[:, pl.ds(0, 128)], ring_v.at[b],
                                  sems[b]).wait()
        pltpu.sync_copy(out_v, out_hbm.at[pl.ds(wid * b_per_w * D,
                                                b_per_w * D)])

    return k(sidx3, dlist3, starts3, ends3, cntb, tab_t)


def _item_lookup(node_ids_item, table_item, B, D, NW, num_cores):
    b_per_w = B // NW
    nchunk = b_per_w // _ICHUNK
    idx3 = jnp.reshape(node_ids_item, (NW, nchunk, _ICHUNK))
    mesh = plsc.VectorSubcoreMesh(core_axis_name="c", subcore_axis_name="s")

    @functools.partial(
        pl.kernel,
        mesh=mesh,
        out_type=jax.ShapeDtypeStruct((B, D), jnp.float32),
        scratch_types=[
            pltpu.VMEM((nchunk, _ICHUNK), jnp.int32),
            pltpu.VMEM((nchunk * _ICHUNK, D), jnp.float32),
            pltpu.SemaphoreType.DMA,
        ],
        compiler_params=pltpu.CompilerParams(use_tc_tiling_on_sc=False),
    )
    def k(idx_hbm, tab_hbm, out_hbm, idx_v, rows_v, sem):
        wid = lax.axis_index("s") * num_cores + lax.axis_index("c")
        pltpu.sync_copy(idx_hbm.at[wid], idx_v)
        copies = [
            pltpu.async_copy(tab_hbm.at[idx_v.at[c]],
                             rows_v.at[pl.ds(c * _ICHUNK, _ICHUNK)], sem)
            for c in range(nchunk)
        ]
        for cp in copies:
            cp.wait()
        base = wid * (nchunk * _ICHUNK)
        pltpu.sync_copy(rows_v, out_hbm.at[pl.ds(base, nchunk * _ICHUNK)])

    return k(idx3, table_item)


def _unpermute(rows_flat, inv, B, D, NW, num_cores):
    b_per_w = B // NW
    nchunk = b_per_w // _ICHUNK
    nvec = b_per_w // 16
    rows2 = jnp.reshape(rows_flat, (B * D // 128, 128))
    pr3 = jnp.reshape(inv >> 1, (NW, nchunk, _ICHUNK))
    h3 = jnp.reshape(inv & 1, (NW, nvec, 16))
    mesh = plsc.VectorSubcoreMesh(core_axis_name="c", subcore_axis_name="s")

    @functools.partial(
        pl.kernel,
        mesh=mesh,
        out_type=jax.ShapeDtypeStruct((D, B), jnp.float32),
        scratch_types=[
            pltpu.VMEM((nchunk, _ICHUNK), jnp.int32),
            pltpu.VMEM((nvec, 16), jnp.int32),
            pltpu.VMEM((b_per_w, 128), jnp.float32),
            pltpu.VMEM((D, b_per_w), jnp.float32),
            pltpu.SemaphoreType.DMA,
        ],
        compiler_params=pltpu.CompilerParams(use_tc_tiling_on_sc=True,
                                             needs_layout_passes=False),
    )
    def k(pr_hbm, h_hbm, rows_hbm, out_hbm, pr_v, h_v, pairs_v, out_v, sem):
        wid = lax.axis_index("s") * num_cores + lax.axis_index("c")
        pltpu.sync_copy(pr_hbm.at[wid], pr_v)
        pltpu.sync_copy(h_hbm.at[wid], h_v)
        copies = [
            pltpu.async_copy(rows_hbm.at[pr_v.at[c]],
                             pairs_v.at[pl.ds(c * _ICHUNK, _ICHUNK)], sem)
            for c in range(nchunk)
        ]
        for cp in copies:
            cp.wait()
        lanes = lax.iota(jnp.int32, 16)

        def gbody(g, carry):
            hv = h_v[g, pl.ds(0, 16)]
            colbase = hv * D
            rowv = lanes + g * 16
            for d in range(D):
                vals = plsc.load_gather(
                    pairs_v, [rowv, colbase + d])
                out_v[d, pl.ds(g * 16, 16)] = vals
            return carry

        lax.fori_loop(0, nvec, gbody, 0)
        out_off = pl.multiple_of(wid * b_per_w, 128)
        pltpu.sync_copy(out_v, out_hbm.at[:, pl.ds(out_off, b_per_w)])

    return k(pr3, h3, rows2)


def kernel(node_ids_user, node_ids_item, table_user, table_item):
    B = node_ids_user.shape[0]
    D = table_user.shape[1]
    info = plsc.get_sparse_core_info()
    NW = info.num_cores * info.num_subcores
    b_per_w = B // NW

    pos = jnp.arange(B, dtype=jnp.int32)
    sidx, spos = lax.sort((node_ids_user, pos), num_keys=1)
    tiles = (sidx >> 7).reshape(NW, b_per_w)
    prev = jnp.concatenate(
        [jnp.full((NW, 1), -1, jnp.int32), tiles[:, :-1]], axis=1)
    newf = tiles != prev
    lrank = jnp.cumsum(newf.astype(jnp.int32), axis=1)
    cnt = lrank[:, -1]
    arange_w = jnp.broadcast_to(jnp.arange(b_per_w, dtype=jnp.int32),
                                (NW, b_per_w))
    ckey = jnp.where(newf, lrank - 1, b_per_w + arange_w)
    start_pay = jnp.where(newf, arange_w, b_per_w)
    _, dlist, starts = lax.sort((ckey, tiles, start_pay), num_keys=1,
                                dimension=1)
    ends = jnp.concatenate(
        [starts[:, 1:], jnp.full((NW, 1), b_per_w, jnp.int32)], axis=1)
    cntb = jnp.broadcast_to(cnt[:, None], (NW, 16))

    tab_t = table_user.T
    rows_flat = _user_lookup_sorted(sidx, dlist, starts, ends, cntb,
                                    tab_t, B, D, NW, info.num_cores)
    spos, rows_flat = lax.optimization_barrier((spos, rows_flat))
    _, inv = lax.sort((spos, pos), num_keys=1)
    emb_u_t = _unpermute(rows_flat, inv, B, D, NW, info.num_cores)
    emb_i = _item_lookup(node_ids_item, table_item, B, D, NW, info.num_cores)
    return (emb_u_t.T, emb_i)


# speedup vs baseline: 1.0264x; 1.0264x over previous
import functools

import jax
import jax.numpy as jnp
from jax import lax
from jax.experimental import pallas as pl
from jax.experimental.pallas import tpu as pltpu, tpu_sc as plsc

_RING = 8
_ICHUNK = 128


def _user_lookup_sorted(sidx, dlist, starts, ends, cntb, tab_t, B, D, NW,
                        num_cores):
    b_per_w = B // NW
    nvec = b_per_w // 16
    sidx3 = jnp.reshape(sidx, (NW, nvec, 16))
    dlist3 = jnp.reshape(dlist, (NW, nvec, 16))
    starts3 = jnp.reshape(starts, (NW, nvec, 16))
    ends3 = jnp.reshape(ends, (NW, nvec, 16))
    ntile_max = (tab_t.shape[1] + 127) // 128 - 1
    mesh = plsc.VectorSubcoreMesh(core_axis_name="c", subcore_axis_name="s")

    @functools.partial(
        pl.kernel,
        mesh=mesh,
        out_type=jax.ShapeDtypeStruct((B * D,), jnp.float32),
        scratch_types=[
            pltpu.VMEM((nvec, 16), jnp.int32),
            pltpu.VMEM((nvec, 16), jnp.int32),
            pltpu.VMEM((nvec, 16), jnp.int32),
            pltpu.VMEM((nvec, 16), jnp.int32),
            pltpu.VMEM((16,), jnp.int32),
            pltpu.VMEM((_RING, D, 128), jnp.float32),
            pltpu.VMEM((b_per_w * D,), jnp.float32),
        ] + [pltpu.SemaphoreType.DMA] * _RING,
        compiler_params=pltpu.CompilerParams(use_tc_tiling_on_sc=True,
                                             needs_layout_passes=False),
    )
    def k(sidx_hbm, dl_hbm, st_hbm, en_hbm, cnt_hbm, tab_hbm, out_hbm,
          sidx_v, dl_v, st_v, en_v, cnt_v, ring_v, out_v, *sems):
        wid = lax.axis_index("s") * num_cores + lax.axis_index("c")
        pltpu.sync_copy(sidx_hbm.at[wid], sidx_v)
        pltpu.sync_copy(dl_hbm.at[wid], dl_v)
        pltpu.sync_copy(st_hbm.at[wid], st_v)
        pltpu.sync_copy(en_hbm.at[wid], en_v)
        pltpu.sync_copy(cnt_hbm.at[wid], cnt_v)
        lanes = lax.iota(jnp.int32, 16)

        def lane_of(vec, lane):
            return jnp.max(jnp.where(lanes == lane, vec, 0))

        def dynload(ref, j):
            vec = ref[j >> 4, pl.ds(0, 16)]
            return lane_of(vec, j & 15)

        cnt = lane_of(cnt_v[pl.ds(0, 16)], 0)

        def fetch(b, d):
            dc = jnp.minimum(d, cnt - 1)
            t = jnp.minimum(dynload(dl_v, dc), ntile_max)
            off = pl.multiple_of(t * 128, 128)
            pltpu.async_copy(tab_hbm.at[:, pl.ds(off, 128)], ring_v.at[b],
                             sems[b])

        for b in range(_RING):
            fetch(b, jnp.int32(b))

        def gbody(g, carry):
            for b in range(_RING):
                d = g * _RING + b
                pltpu.make_async_copy(tab_hbm.at[:, pl.ds(0, 128)],
                                      ring_v.at[b], sems[b]).wait()
                dc = jnp.minimum(d, cnt - 1)
                s = dynload(st_v, dc)
                e = dynload(en_v, dc)

                def ebody(j, car):
                    iv = dynload(sidx_v, j)
                    col = iv & 127
                    for kq in range(D // 16):
                        dvec = lanes + kq * 16
                        vals = plsc.load_gather(
                            ring_v.at[b],
                            [dvec, jnp.full((16,), 1, jnp.int32) * col])
                        plsc.store_scatter(
                            out_v, [lanes + (j * D + kq * 16)], vals)
                    return car

                lax.fori_loop(s, e, ebody, 0)
                fetch(b, (g + 1) * _RING + b)
            return carry

        ngroups = (cnt + _RING - 1) >> 3
        lax.fori_loop(0, ngroups, gbody, 0)
        for b in range(_RING):
            pltpu.make_async_copy(tab_hbm.at[:, pl.ds(0, 128)], ring_v.at[b],
                                  sems[b]).wait()
        pltpu.sync_copy(out_v, out_hbm.at[pl.ds(wid * b_per_w * D,
                                                b_per_w * D)])

    return k(sidx3, dlist3, starts3, ends3, cntb, tab_t)


def _item_and_unpermute(node_ids_item, table_item, rows_sorted, inv, B, D, NW,
                        num_cores):
    b_per_w = B // NW
    nchunk = b_per_w // _ICHUNK
    idx3 = jnp.reshape(node_ids_item, (NW, nchunk, _ICHUNK))
    inv3 = jnp.reshape(inv, (NW, nchunk, _ICHUNK))
    mesh = plsc.VectorSubcoreMesh(core_axis_name="c", subcore_axis_name="s")

    @functools.partial(
        pl.kernel,
        mesh=mesh,
        out_type=(
            jax.ShapeDtypeStruct((NW, nchunk, _ICHUNK, D), jnp.float32),
            jax.ShapeDtypeStruct((NW, nchunk, _ICHUNK, D), jnp.float32),
        ),
        scratch_types=[
            pltpu.VMEM((nchunk, _ICHUNK), jnp.int32),
            pltpu.VMEM((nchunk, _ICHUNK), jnp.int32),
            pltpu.VMEM((nchunk, _ICHUNK, D), jnp.float32),
            pltpu.VMEM((nchunk, _ICHUNK, D), jnp.float32),
            pltpu.SemaphoreType.DMA,
            pltpu.SemaphoreType.DMA,
        ],
        compiler_params=pltpu.CompilerParams(use_tc_tiling_on_sc=False),
    )
    def k(idx_hbm, inv_hbm, tab_hbm, rows_hbm, out_i_hbm, out_u_hbm,
          idx_v, inv_v, rows_i_v, rows_u_v, sem_i, sem_u):
        wid = lax.axis_index("s") * num_cores + lax.axis_index("c")
        pltpu.sync_copy(idx_hbm.at[wid], idx_v)
        pltpu.sync_copy(inv_hbm.at[wid], inv_v)
        copies = []
        for c in range(nchunk):
            copies.append(
                pltpu.async_copy(tab_hbm.at[idx_v.at[c]], rows_i_v.at[c],
                                 sem_i))
            copies.append(
                pltpu.async_copy(rows_hbm.at[inv_v.at[c]], rows_u_v.at[c],
                                 sem_u))
        for cp in copies:
            cp.wait()
        pltpu.sync_copy(rows_i_v, out_i_hbm.at[wid])
        pltpu.sync_copy(rows_u_v, out_u_hbm.at[wid])

    out_i, out_u = k(idx3, inv3, table_item, rows_sorted)
    return jnp.reshape(out_i, (B, D)), jnp.reshape(out_u, (B, D))


def kernel(node_ids_user, node_ids_item, table_user, table_item):
    B = node_ids_user.shape[0]
    D = table_user.shape[1]
    info = plsc.get_sparse_core_info()
    NW = info.num_cores * info.num_subcores
    b_per_w = B // NW

    pos = jnp.arange(B, dtype=jnp.int32)
    sidx, spos = lax.sort((node_ids_user, pos), num_keys=1)
    tiles = (sidx >> 7).reshape(NW, b_per_w)
    prev = jnp.concatenate(
        [jnp.full((NW, 1), -1, jnp.int32), tiles[:, :-1]], axis=1)
    newf = tiles != prev
    lrank = jnp.cumsum(newf.astype(jnp.int32), axis=1)
    cnt = lrank[:, -1]
    arange_w = jnp.broadcast_to(jnp.arange(b_per_w, dtype=jnp.int32),
                                (NW, b_per_w))
    ckey = jnp.where(newf, lrank - 1, b_per_w + arange_w)
    start_pay = jnp.where(newf, arange_w, b_per_w)
    _, dlist, starts = lax.sort((ckey, tiles, start_pay), num_keys=1,
                                dimension=1)
    ends = jnp.concatenate(
        [starts[:, 1:], jnp.full((NW, 1), b_per_w, jnp.int32)], axis=1)
    cntb = jnp.broadcast_to(cnt[:, None], (NW, 16))
    _, inv = lax.sort((spos, pos), num_keys=1)

    tab_t = table_user.T
    rows_flat = _user_lookup_sorted(sidx, dlist, starts, ends, cntb,
                                    tab_t, B, D, NW, info.num_cores)
    emb_i, emb_u = _item_and_unpermute(node_ids_item, table_item,
                                       jnp.reshape(rows_flat, (B, D)), inv,
                                       B, D, NW, info.num_cores)
    return (emb_u, emb_i)
